# baseline (speedup 1.0000x reference)
import math

import jax
import jax.numpy as jnp
import numpy as np
from jax.experimental import pallas as pl
from jax.experimental.pallas import tpu as pltpu

_F32 = jnp.float32
_BF16 = jnp.bfloat16


def _site_ln(z, g, b, eps=1e-6):
    mu = jnp.mean(z, axis=-1, keepdims=True)
    var = jnp.mean((z - mu) ** 2, axis=-1, keepdims=True)
    return (z - mu) * jax.lax.rsqrt(var + eps) * g + b


def _gelu(x):
    return 0.5 * x * (1.0 + jax.lax.erf(x * np.float32(1.0 / math.sqrt(2.0))))


def _stem_kernel(p_ref, keep_ref, w_ref, b_ref, g_ref, gb_ref, o_ref):
    B, H, W, K = p_ref.shape
    C = w_ref.shape[1]
    p = p_ref[...].reshape(B * H * W, K)
    h = jnp.dot(p.astype(_BF16), w_ref[...],
                preferred_element_type=_F32) + b_ref[...]
    h = _site_ln(h, g_ref[...], gb_ref[...])
    o_ref[...] = h.reshape(B, H, W, C) * keep_ref[...][..., None]


def _block_kernel(y_ref, keep_ref, dww_ref, dwb_ref, lng_ref, lnb_ref,
                  w1_ref, b1_ref, gg_ref, gb_ref, w2_ref, b2_ref,
                  o_ref, pad_ref):
    B, H, W, C = y_ref.shape
    C4 = w1_ref.shape[1]
    keep4 = keep_ref[...][..., None]
    pad_ref[...] = jnp.zeros(pad_ref.shape, _F32)
    y = y_ref[...]
    pad_ref[:, 3:H + 3, 3:W + 3, :] = y
    acc = jnp.zeros((B, H, W, C), _F32)
    for k in range(49):
        dh, dw = divmod(k, 7)
        acc += pad_ref[:, dh:dh + H, dw:dw + W, :] * dww_ref[k:k + 1, :]
    z = (acc + dwb_ref[...]) * keep4
    z = _site_ln(z, lng_ref[...], lnb_ref[...])
    h = jnp.dot(z.reshape(B * H * W, C).astype(_BF16), w1_ref[...],
                preferred_element_type=_F32) + b1_ref[...]
    h = _gelu(h)
    h3 = h.reshape(B, H * W, C4)
    gx = jnp.sqrt(jnp.sum(h3 * h3, axis=1, keepdims=True))
    nx = gx / (jnp.mean(gx, axis=-1, keepdims=True) + 1e-6)
    h3 = gg_ref[...] * (h3 * nx) + gb_ref[...] + h3
    o = jnp.dot(h3.reshape(B * H * W, C4).astype(_BF16), w2_ref[...],
                preferred_element_type=_F32) + b2_ref[...]
    o_ref[...] = o.reshape(B, H, W, C) * keep4 + y


def _down_kernel(p_ref, kp_ref, g_ref, b_ref, w_ref, wb_ref, kc_ref, o_ref):
    B, H2, W2, C4 = p_ref.shape
    C = C4 // 4
    O = w_ref.shape[1]
    p = p_ref[...]
    kp = kp_ref[...]
    acc = jnp.zeros((B * H2 * W2, O), _F32)
    for k in range(4):
        zk = _site_ln(p[..., k * C:(k + 1) * C], g_ref[...], b_ref[...])
        zk = zk * kp[..., k:k + 1]
        acc += jnp.dot(zk.reshape(B * H2 * W2, C).astype(_BF16),
                       w_ref[k * C:(k + 1) * C, :],
                       preferred_element_type=_F32)
    o = (acc + wb_ref[...]).reshape(B, H2, W2, O) * kc_ref[...][..., None]
    o_ref[...] = o


def _stem_call(p, keep, w, b, g, gb):
    B, H, W, _ = p.shape
    C = w.shape[1]
    return pl.pallas_call(
        _stem_kernel,
        out_shape=jax.ShapeDtypeStruct((B, H, W, C), _F32),
    )(p, keep, w, b, g, gb)


def _block_call(y, keep, dww, dwb, lng, lnb, w1, b1, gg, gb, w2, b2):
    B, H, W, C = y.shape
    return pl.pallas_call(
        _block_kernel,
        out_shape=jax.ShapeDtypeStruct((B, H, W, C), _F32),
        scratch_shapes=[pltpu.VMEM((B, H + 6, W + 6, C), _F32)],
    )(y, keep, dww, dwb, lng, lnb, w1, b1, gg, gb, w2, b2)


def _down_call(p, kp, g, b, w, wb, kc):
    B, H2, W2, _ = p.shape
    O = w.shape[1]
    return pl.pallas_call(
        _down_kernel,
        out_shape=jax.ShapeDtypeStruct((B, H2, W2, O), _F32),
    )(p, kp, g, b, w, wb, kc)


def _row(v):
    return jnp.asarray(v, _F32).reshape(1, -1)


def kernel(x, mask, params):
    B = x.shape[0]
    m = mask.reshape(B, 7, 7)
    m = jnp.repeat(jnp.repeat(m, 8, axis=1), 8, axis=2)
    keep0 = (1 - m).astype(_F32)
    keeps = [keep0, keep0[:, ::2, ::2], keep0[:, ::4, ::4], keep0[:, ::8, ::8]]

    p = x.reshape(B, 3, 56, 4, 56, 4).transpose(0, 2, 4, 1, 3, 5)
    p = p.reshape(B, 56, 56, 48)
    wf = params['stem_w'].transpose(1, 2, 3, 0).reshape(48, -1).astype(_BF16)
    y = _stem_call(p, keeps[0], wf, _row(params['stem_b']),
                   _row(params['stem_ln_g']), _row(params['stem_ln_b']))

    for i in range(4):
        if i > 0:
            dp = params['downs'][i - 1]
            H, C = y.shape[1], y.shape[3]
            yp = y.reshape(B, H // 2, 2, H // 2, 2, C)
            yp = yp.transpose(0, 1, 3, 2, 4, 5).reshape(B, H // 2, H // 2, 4 * C)
            kf = keeps[i - 1].reshape(B, H // 2, 2, H // 2, 2)
            kf = kf.transpose(0, 1, 3, 2, 4).reshape(B, H // 2, H // 2, 4)
            wt = dp['w'].transpose(2, 3, 1, 0).reshape(4 * C, -1).astype(_BF16)
            y = _down_call(yp, kf, _row(dp['ln_g']), _row(dp['ln_b']),
                           wt, _row(dp['b']), keeps[i])
        for bp in params['stages'][i]:
            C = y.shape[3]
            dww = bp['dw_w'][:, 0].transpose(1, 2, 0).reshape(49, C)
            y = _block_call(
                y, keeps[i], dww, _row(bp['dw_b']), _row(bp['ln_g']),
                _row(bp['ln_b']), bp['w1'].astype(_BF16), _row(bp['b1']),
                bp['grn_g'].reshape(1, -1), bp['grn_b'].reshape(1, -1),
                bp['w2'].astype(_BF16), _row(bp['b2']))

    return y.transpose(0, 3, 1, 2)


# speedup vs baseline: 1.1660x; 1.1660x over previous
import math

import jax
import jax.numpy as jnp
import numpy as np
from jax.experimental import pallas as pl
from jax.experimental.pallas import tpu as pltpu

_F32 = jnp.float32
_BF16 = jnp.bfloat16


def _site_ln(z, g, b, eps=1e-6):
    mu = jnp.mean(z, axis=-1, keepdims=True)
    var = jnp.mean((z - mu) ** 2, axis=-1, keepdims=True)
    return (z - mu) * jax.lax.rsqrt(var + eps) * g + b


def _gelu(x):
    return 0.5 * x * (1.0 + jax.lax.erf(x * np.float32(1.0 / math.sqrt(2.0))))


def _stem_kernel(p_ref, keep_ref, w_ref, b_ref, g_ref, gb_ref, o_ref):
    B, H, W, K = p_ref.shape
    C = w_ref.shape[1]
    p = p_ref[...].reshape(B * H * W, K)
    h = jnp.dot(p.astype(_BF16), w_ref[...],
                preferred_element_type=_F32) + b_ref[...]
    h = _site_ln(h, g_ref[...], gb_ref[...])
    o_ref[...] = h.reshape(B, H, W, C) * keep_ref[...][..., None]


def _stage_kernel(y_ref, keep_ref, dww_ref, dwb_ref, lng_ref, lnb_ref,
                  w1_ref, b1_ref, gg_ref, gb_ref, w2_ref, b2_ref,
                  o_ref, pad_ref):
    B, H, W, C = y_ref.shape
    C4 = w1_ref.shape[2]
    j = pl.program_id(0)

    @pl.when(j == 0)
    def _init():
        o_ref[...] = y_ref[...]

    keep4 = keep_ref[...][..., None]
    pad_ref[...] = jnp.zeros(pad_ref.shape, _F32)
    y = o_ref[...]
    pad_ref[:, 3:H + 3, 3:W + 3, :] = y
    acc = jnp.zeros((B, H, W, C), _F32)
    for k in range(49):
        dh, dw = divmod(k, 7)
        acc += pad_ref[:, dh:dh + H, dw:dw + W, :] * dww_ref[0, k:k + 1, :]
    z = (acc + dwb_ref[0]) * keep4
    z = _site_ln(z, lng_ref[0], lnb_ref[0])
    h = jnp.dot(z.reshape(B * H * W, C).astype(_BF16), w1_ref[0],
                preferred_element_type=_F32) + b1_ref[0]
    h = _gelu(h)
    h3 = h.reshape(B, H * W, C4)
    gx = jnp.sqrt(jnp.sum(h3 * h3, axis=1, keepdims=True))
    nx = gx / (jnp.mean(gx, axis=-1, keepdims=True) + 1e-6)
    h3 = gg_ref[0] * (h3 * nx) + gb_ref[0] + h3
    o = jnp.dot(h3.reshape(B * H * W, C4).astype(_BF16), w2_ref[0],
                preferred_element_type=_F32) + b2_ref[0]
    o_ref[...] = o.reshape(B, H, W, C) * keep4 + y


def _down_kernel(p_ref, kp_ref, g_ref, b_ref, w_ref, wb_ref, kc_ref, o_ref):
    B, H2, W2, C4 = p_ref.shape
    C = C4 // 4
    O = w_ref.shape[1]
    p = p_ref[...]
    kp = kp_ref[...]
    acc = jnp.zeros((B * H2 * W2, O), _F32)
    for k in range(4):
        zk = _site_ln(p[..., k * C:(k + 1) * C], g_ref[...], b_ref[...])
        zk = zk * kp[..., k:k + 1]
        acc += jnp.dot(zk.reshape(B * H2 * W2, C).astype(_BF16),
                       w_ref[k * C:(k + 1) * C, :],
                       preferred_element_type=_F32)
    o = (acc + wb_ref[...]).reshape(B, H2, W2, O) * kc_ref[...][..., None]
    o_ref[...] = o


def _stem_call(p, keep, w, b, g, gb):
    B, H, W, _ = p.shape
    C = w.shape[1]
    return pl.pallas_call(
        _stem_kernel,
        out_shape=jax.ShapeDtypeStruct((B, H, W, C), _F32),
    )(p, keep, w, b, g, gb)


def _stage_call(y, keep, dww, dwb, lng, lnb, w1, b1, gg, gb, w2, b2):
    B, H, W, C = y.shape
    nb, C4 = w1.shape[0], w1.shape[2]
    wspec = lambda s1, s2: pl.BlockSpec((1, s1, s2), lambda j: (j, 0, 0))
    return pl.pallas_call(
        _stage_kernel,
        grid=(nb,),
        in_specs=[
            pl.BlockSpec((B, H, W, C), lambda j: (0, 0, 0, 0)),
            pl.BlockSpec((B, H, W), lambda j: (0, 0, 0)),
            wspec(49, C), wspec(1, C), wspec(1, C), wspec(1, C),
            wspec(C, C4), wspec(1, C4), wspec(1, C4), wspec(1, C4),
            wspec(C4, C), wspec(1, C),
        ],
        out_specs=pl.BlockSpec((B, H, W, C), lambda j: (0, 0, 0, 0)),
        out_shape=jax.ShapeDtypeStruct((B, H, W, C), _F32),
        scratch_shapes=[pltpu.VMEM((B, H + 6, W + 6, C), _F32)],
    )(y, keep, dww, dwb, lng, lnb, w1, b1, gg, gb, w2, b2)


def _down_call(p, kp, g, b, w, wb, kc):
    B, H2, W2, _ = p.shape
    O = w.shape[1]
    return pl.pallas_call(
        _down_kernel,
        out_shape=jax.ShapeDtypeStruct((B, H2, W2, O), _F32),
    )(p, kp, g, b, w, wb, kc)


def _row(v):
    return jnp.asarray(v, _F32).reshape(1, -1)


def kernel(x, mask, params):
    B = x.shape[0]
    m = mask.reshape(B, 7, 7)
    m = jnp.repeat(jnp.repeat(m, 8, axis=1), 8, axis=2)
    keep0 = (1 - m).astype(_F32)
    keeps = [keep0, keep0[:, ::2, ::2], keep0[:, ::4, ::4], keep0[:, ::8, ::8]]

    p = x.reshape(B, 3, 56, 4, 56, 4).transpose(0, 2, 4, 1, 3, 5)
    p = p.reshape(B, 56, 56, 48)
    wf = params['stem_w'].transpose(1, 2, 3, 0).reshape(48, -1).astype(_BF16)
    y = _stem_call(p, keeps[0], wf, _row(params['stem_b']),
                   _row(params['stem_ln_g']), _row(params['stem_ln_b']))

    for i in range(4):
        if i > 0:
            dp = params['downs'][i - 1]
            H, C = y.shape[1], y.shape[3]
            yp = y.reshape(B, H // 2, 2, H // 2, 2, C)
            yp = yp.transpose(0, 1, 3, 2, 4, 5).reshape(B, H // 2, H // 2, 4 * C)
            kf = keeps[i - 1].reshape(B, H // 2, 2, H // 2, 2)
            kf = kf.transpose(0, 1, 3, 2, 4).reshape(B, H // 2, H // 2, 4)
            wt = dp['w'].transpose(2, 3, 1, 0).reshape(4 * C, -1).astype(_BF16)
            y = _down_call(yp, kf, _row(dp['ln_g']), _row(dp['ln_b']),
                           wt, _row(dp['b']), keeps[i])
        blocks = params['stages'][i]
        C = y.shape[3]
        dww = jnp.stack([bp['dw_w'][:, 0].transpose(1, 2, 0).reshape(49, C)
                         for bp in blocks])
        dwb = jnp.stack([_row(bp['dw_b']) for bp in blocks])
        lng = jnp.stack([_row(bp['ln_g']) for bp in blocks])
        lnb = jnp.stack([_row(bp['ln_b']) for bp in blocks])
        w1 = jnp.stack([bp['w1'] for bp in blocks]).astype(_BF16)
        b1 = jnp.stack([_row(bp['b1']) for bp in blocks])
        gg = jnp.stack([bp['grn_g'].reshape(1, -1) for bp in blocks])
        gb = jnp.stack([bp['grn_b'].reshape(1, -1) for bp in blocks])
        w2 = jnp.stack([bp['w2'] for bp in blocks]).astype(_BF16)
        b2 = jnp.stack([_row(bp['b2']) for bp in blocks])
        y = _stage_call(y, keeps[i], dww, dwb, lng, lnb,
                        w1, b1, gg, gb, w2, b2)

    return y.transpose(0, 3, 1, 2)


# speedup vs baseline: 1.2088x; 1.0367x over previous
import math

import jax
import jax.numpy as jnp
import numpy as np
from jax.experimental import pallas as pl
from jax.experimental.pallas import tpu as pltpu

_F32 = jnp.float32
_BF16 = jnp.bfloat16


def _site_ln(z, g, b, eps=1e-6):
    mu = jnp.mean(z, axis=-1, keepdims=True)
    var = jnp.mean((z - mu) ** 2, axis=-1, keepdims=True)
    return (z - mu) * jax.lax.rsqrt(var + eps) * g + b


def _gelu(x):
    return 0.5 * x * (1.0 + jax.lax.erf(x * np.float32(1.0 / math.sqrt(2.0))))


def _stem_kernel(p_ref, keep_ref, w_ref, b_ref, g_ref, gb_ref, o_ref):
    B, H, W, K = p_ref.shape
    C = w_ref.shape[1]
    p = p_ref[...].reshape(B * H * W, K)
    h = jnp.dot(p.astype(_BF16), w_ref[...],
                preferred_element_type=_F32) + b_ref[...]
    h = _site_ln(h, g_ref[...], gb_ref[...])
    o_ref[...] = h.reshape(B, H, W, C) * keep_ref[...][..., None]


def _stage_kernel(y_ref, keep_ref, dww_ref, dwb_ref, lng_ref, lnb_ref,
                  w1_ref, b1_ref, gg_ref, gb_ref, w2_ref, b2_ref,
                  o_ref, pad_ref):
    B, H, W, C = y_ref.shape
    C4 = w1_ref.shape[2]
    j = pl.program_id(0)

    @pl.when(j == 0)
    def _init():
        o_ref[...] = y_ref[...]
        pad_ref[...] = jnp.zeros(pad_ref.shape, _BF16)

    keep4 = keep_ref[...][..., None]
    y = o_ref[...]
    pad_ref[:, 3:H + 3, 3:W + 3, :] = y.astype(_BF16)
    acc = jnp.zeros((B, H, W, C), _F32)
    for k in range(49):
        dh, dw = divmod(k, 7)
        acc += pad_ref[:, dh:dh + H, dw:dw + W, :].astype(_F32) * dww_ref[0, k:k + 1, :]
    z = (acc + dwb_ref[0]) * keep4
    z = _site_ln(z, lng_ref[0], lnb_ref[0])
    h = jnp.dot(z.reshape(B * H * W, C).astype(_BF16), w1_ref[0],
                preferred_element_type=_F32) + b1_ref[0]
    h = _gelu(h)
    h3 = h.reshape(B, H * W, C4)
    gx = jnp.sqrt(jnp.sum(h3 * h3, axis=1, keepdims=True))
    nx = gx / (jnp.mean(gx, axis=-1, keepdims=True) + 1e-6)
    h3 = gg_ref[0] * (h3 * nx) + gb_ref[0] + h3
    o = jnp.dot(h3.reshape(B * H * W, C4).astype(_BF16), w2_ref[0],
                preferred_element_type=_F32) + b2_ref[0]
    o_ref[...] = o.reshape(B, H, W, C) * keep4 + y


def _down_kernel(p_ref, kp_ref, g_ref, b_ref, w_ref, wb_ref, kc_ref, o_ref):
    B, H2, W2, C4 = p_ref.shape
    C = C4 // 4
    O = w_ref.shape[1]
    p = p_ref[...]
    kp = kp_ref[...]
    acc = jnp.zeros((B * H2 * W2, O), _F32)
    for k in range(4):
        zk = _site_ln(p[..., k * C:(k + 1) * C], g_ref[...], b_ref[...])
        zk = zk * kp[..., k:k + 1]
        acc += jnp.dot(zk.reshape(B * H2 * W2, C).astype(_BF16),
                       w_ref[k * C:(k + 1) * C, :],
                       preferred_element_type=_F32)
    o = (acc + wb_ref[...]).reshape(B, H2, W2, O) * kc_ref[...][..., None]
    o_ref[...] = o


def _stem_call(p, keep, w, b, g, gb):
    B, H, W, _ = p.shape
    C = w.shape[1]
    return pl.pallas_call(
        _stem_kernel,
        out_shape=jax.ShapeDtypeStruct((B, H, W, C), _F32),
    )(p, keep, w, b, g, gb)


def _stage_call(y, keep, dww, dwb, lng, lnb, w1, b1, gg, gb, w2, b2):
    B, H, W, C = y.shape
    nb, C4 = w1.shape[0], w1.shape[2]
    wspec = lambda s1, s2: pl.BlockSpec((1, s1, s2), lambda j: (j, 0, 0))
    return pl.pallas_call(
        _stage_kernel,
        grid=(nb,),
        in_specs=[
            pl.BlockSpec((B, H, W, C), lambda j: (0, 0, 0, 0)),
            pl.BlockSpec((B, H, W), lambda j: (0, 0, 0)),
            wspec(49, C), wspec(1, C), wspec(1, C), wspec(1, C),
            wspec(C, C4), wspec(1, C4), wspec(1, C4), wspec(1, C4),
            wspec(C4, C), wspec(1, C),
        ],
        out_specs=pl.BlockSpec((B, H, W, C), lambda j: (0, 0, 0, 0)),
        out_shape=jax.ShapeDtypeStruct((B, H, W, C), _F32),
        scratch_shapes=[pltpu.VMEM((B, H + 6, W + 6, C), _BF16)],
    )(y, keep, dww, dwb, lng, lnb, w1, b1, gg, gb, w2, b2)


def _down_call(p, kp, g, b, w, wb, kc):
    B, H2, W2, _ = p.shape
    O = w.shape[1]
    return pl.pallas_call(
        _down_kernel,
        out_shape=jax.ShapeDtypeStruct((B, H2, W2, O), _F32),
    )(p, kp, g, b, w, wb, kc)


def _row(v):
    return jnp.asarray(v, _F32).reshape(1, -1)


def kernel(x, mask, params):
    B = x.shape[0]
    m = mask.reshape(B, 7, 7)
    m = jnp.repeat(jnp.repeat(m, 8, axis=1), 8, axis=2)
    keep0 = (1 - m).astype(_F32)
    keeps = [keep0, keep0[:, ::2, ::2], keep0[:, ::4, ::4], keep0[:, ::8, ::8]]

    p = x.reshape(B, 3, 56, 4, 56, 4).transpose(0, 2, 4, 1, 3, 5)
    p = p.reshape(B, 56, 56, 48)
    wf = params['stem_w'].transpose(1, 2, 3, 0).reshape(48, -1).astype(_BF16)
    y = _stem_call(p, keeps[0], wf, _row(params['stem_b']),
                   _row(params['stem_ln_g']), _row(params['stem_ln_b']))

    for i in range(4):
        if i > 0:
            dp = params['downs'][i - 1]
            H, C = y.shape[1], y.shape[3]
            yp = y.reshape(B, H // 2, 2, H // 2, 2, C)
            yp = yp.transpose(0, 1, 3, 2, 4, 5).reshape(B, H // 2, H // 2, 4 * C)
            kf = keeps[i - 1].reshape(B, H // 2, 2, H // 2, 2)
            kf = kf.transpose(0, 1, 3, 2, 4).reshape(B, H // 2, H // 2, 4)
            wt = dp['w'].transpose(2, 3, 1, 0).reshape(4 * C, -1).astype(_BF16)
            y = _down_call(yp, kf, _row(dp['ln_g']), _row(dp['ln_b']),
                           wt, _row(dp['b']), keeps[i])
        blocks = params['stages'][i]
        C = y.shape[3]
        dww = jnp.stack([bp['dw_w'][:, 0].transpose(1, 2, 0).reshape(49, C)
                         for bp in blocks])
        dwb = jnp.stack([_row(bp['dw_b']) for bp in blocks])
        lng = jnp.stack([_row(bp['ln_g']) for bp in blocks])
        lnb = jnp.stack([_row(bp['ln_b']) for bp in blocks])
        w1 = jnp.stack([bp['w1'] for bp in blocks]).astype(_BF16)
        b1 = jnp.stack([_row(bp['b1']) for bp in blocks])
        gg = jnp.stack([bp['grn_g'].reshape(1, -1) for bp in blocks])
        gb = jnp.stack([bp['grn_b'].reshape(1, -1) for bp in blocks])
        w2 = jnp.stack([bp['w2'] for bp in blocks]).astype(_BF16)
        b2 = jnp.stack([_row(bp['b2']) for bp in blocks])
        y = _stage_call(y, keeps[i], dww, dwb, lng, lnb,
                        w1, b1, gg, gb, w2, b2)

    return y.transpose(0, 3, 1, 2)


# speedup vs baseline: 1.2095x; 1.0006x over previous
import math

import jax
import jax.numpy as jnp
import numpy as np
from jax.experimental import pallas as pl
from jax.experimental.pallas import tpu as pltpu

_F32 = jnp.float32
_BF16 = jnp.bfloat16


def _site_ln(z, g, b, eps=1e-6):
    mu = jnp.mean(z, axis=-1, keepdims=True)
    var = jnp.mean((z - mu) ** 2, axis=-1, keepdims=True)
    return (z - mu) * jax.lax.rsqrt(var + eps) * g + b


def _gelu(x):
    return 0.5 * x * (1.0 + jax.lax.erf(x * np.float32(1.0 / math.sqrt(2.0))))


def _stem_kernel(p_ref, keep_ref, w_ref, b_ref, g_ref, gb_ref, o_ref):
    B, H, W, K = p_ref.shape
    C = w_ref.shape[1]
    p = p_ref[...].reshape(B * H * W, K)
    h = jnp.dot(p.astype(_BF16), w_ref[...],
                preferred_element_type=_F32) + b_ref[...]
    h = _site_ln(h, g_ref[...], gb_ref[...])
    o_ref[...] = h.reshape(B, H, W, C) * keep_ref[...][..., None]


def _stage_kernel(y_ref, keep_ref, dww_ref, dwb_ref, lng_ref, lnb_ref,
                  w1_ref, b1_ref, gg_ref, gb_ref, w2_ref, b2_ref,
                  o_ref, pad_ref):
    B, H, W, C = y_ref.shape
    C4 = w1_ref.shape[2]
    j = pl.program_id(0)

    @pl.when(j == 0)
    def _init():
        o_ref[...] = y_ref[...]
        pad_ref[...] = jnp.zeros(pad_ref.shape, _BF16)

    keep4 = keep_ref[...][..., None]
    y = o_ref[...]
    yb = y.astype(_BF16)
    for dw in range(7):
        s = dw - 3
        a, b = max(0, -s), W - max(0, s)
        pad_ref[dw, :, 3:H + 3, a:b, :] = yb[:, :, a + s:b + s, :]
    acc = jnp.zeros((B, H, W, C), _F32)
    for k in range(49):
        dh, dw = divmod(k, 7)
        acc += (pad_ref[dw, :, dh:dh + H, :, :].astype(_F32)
                * dww_ref[0, k:k + 1, :])
    z = (acc + dwb_ref[0]) * keep4
    z = _site_ln(z, lng_ref[0], lnb_ref[0])
    h = jnp.dot(z.reshape(B * H * W, C).astype(_BF16), w1_ref[0],
                preferred_element_type=_F32) + b1_ref[0]
    h = _gelu(h)
    h3 = h.reshape(B, H * W, C4)
    gx = jnp.sqrt(jnp.sum(h3 * h3, axis=1, keepdims=True))
    nx = gx / (jnp.mean(gx, axis=-1, keepdims=True) + 1e-6)
    h3 = gg_ref[0] * (h3 * nx) + gb_ref[0] + h3
    o = jnp.dot(h3.reshape(B * H * W, C4).astype(_BF16), w2_ref[0],
                preferred_element_type=_F32) + b2_ref[0]
    o_ref[...] = o.reshape(B, H, W, C) * keep4 + y


def _down_kernel(p_ref, kp_ref, g_ref, b_ref, w_ref, wb_ref, kc_ref, o_ref):
    B, H2, W2, C4 = p_ref.shape
    C = C4 // 4
    O = w_ref.shape[1]
    p = p_ref[...]
    kp = kp_ref[...]
    acc = jnp.zeros((B * H2 * W2, O), _F32)
    for k in range(4):
        zk = _site_ln(p[..., k * C:(k + 1) * C], g_ref[...], b_ref[...])
        zk = zk * kp[..., k:k + 1]
        acc += jnp.dot(zk.reshape(B * H2 * W2, C).astype(_BF16),
                       w_ref[k * C:(k + 1) * C, :],
                       preferred_element_type=_F32)
    o = (acc + wb_ref[...]).reshape(B, H2, W2, O) * kc_ref[...][..., None]
    o_ref[...] = o


def _stem_call(p, keep, w, b, g, gb):
    B, H, W, _ = p.shape
    C = w.shape[1]
    return pl.pallas_call(
        _stem_kernel,
        out_shape=jax.ShapeDtypeStruct((B, H, W, C), _F32),
    )(p, keep, w, b, g, gb)


def _stage_call(y, keep, dww, dwb, lng, lnb, w1, b1, gg, gb, w2, b2):
    B, H, W, C = y.shape
    nb, C4 = w1.shape[0], w1.shape[2]
    wspec = lambda s1, s2: pl.BlockSpec((1, s1, s2), lambda j: (j, 0, 0))
    return pl.pallas_call(
        _stage_kernel,
        grid=(nb,),
        in_specs=[
            pl.BlockSpec((B, H, W, C), lambda j: (0, 0, 0, 0)),
            pl.BlockSpec((B, H, W), lambda j: (0, 0, 0)),
            wspec(49, C), wspec(1, C), wspec(1, C), wspec(1, C),
            wspec(C, C4), wspec(1, C4), wspec(1, C4), wspec(1, C4),
            wspec(C4, C), wspec(1, C),
        ],
        out_specs=pl.BlockSpec((B, H, W, C), lambda j: (0, 0, 0, 0)),
        out_shape=jax.ShapeDtypeStruct((B, H, W, C), _F32),
        scratch_shapes=[pltpu.VMEM((7, B, H + 6, W, C), _BF16)],
    )(y, keep, dww, dwb, lng, lnb, w1, b1, gg, gb, w2, b2)


def _down_call(p, kp, g, b, w, wb, kc):
    B, H2, W2, _ = p.shape
    O = w.shape[1]
    return pl.pallas_call(
        _down_kernel,
        out_shape=jax.ShapeDtypeStruct((B, H2, W2, O), _F32),
    )(p, kp, g, b, w, wb, kc)


def _row(v):
    return jnp.asarray(v, _F32).reshape(1, -1)


def kernel(x, mask, params):
    B = x.shape[0]
    m = mask.reshape(B, 7, 7)
    m = jnp.repeat(jnp.repeat(m, 8, axis=1), 8, axis=2)
    keep0 = (1 - m).astype(_F32)
    keeps = [keep0, keep0[:, ::2, ::2], keep0[:, ::4, ::4], keep0[:, ::8, ::8]]

    p = x.reshape(B, 3, 56, 4, 56, 4).transpose(0, 2, 4, 1, 3, 5)
    p = p.reshape(B, 56, 56, 48)
    wf = params['stem_w'].transpose(1, 2, 3, 0).reshape(48, -1).astype(_BF16)
    y = _stem_call(p, keeps[0], wf, _row(params['stem_b']),
                   _row(params['stem_ln_g']), _row(params['stem_ln_b']))

    for i in range(4):
        if i > 0:
            dp = params['downs'][i - 1]
            H, C = y.shape[1], y.shape[3]
            yp = y.reshape(B, H // 2, 2, H // 2, 2, C)
            yp = yp.transpose(0, 1, 3, 2, 4, 5).reshape(B, H // 2, H // 2, 4 * C)
            kf = keeps[i - 1].reshape(B, H // 2, 2, H // 2, 2)
            kf = kf.transpose(0, 1, 3, 2, 4).reshape(B, H // 2, H // 2, 4)
            wt = dp['w'].transpose(2, 3, 1, 0).reshape(4 * C, -1).astype(_BF16)
            y = _down_call(yp, kf, _row(dp['ln_g']), _row(dp['ln_b']),
                           wt, _row(dp['b']), keeps[i])
        blocks = params['stages'][i]
        C = y.shape[3]
        dww = jnp.stack([bp['dw_w'][:, 0].transpose(1, 2, 0).reshape(49, C)
                         for bp in blocks])
        dwb = jnp.stack([_row(bp['dw_b']) for bp in blocks])
        lng = jnp.stack([_row(bp['ln_g']) for bp in blocks])
        lnb = jnp.stack([_row(bp['ln_b']) for bp in blocks])
        w1 = jnp.stack([bp['w1'] for bp in blocks]).astype(_BF16)
        b1 = jnp.stack([_row(bp['b1']) for bp in blocks])
        gg = jnp.stack([bp['grn_g'].reshape(1, -1) for bp in blocks])
        gb = jnp.stack([bp['grn_b'].reshape(1, -1) for bp in blocks])
        w2 = jnp.stack([bp['w2'] for bp in blocks]).astype(_BF16)
        b2 = jnp.stack([_row(bp['b2']) for bp in blocks])
        y = _stage_call(y, keeps[i], dww, dwb, lng, lnb,
                        w1, b1, gg, gb, w2, b2)

    return y.transpose(0, 3, 1, 2)
